# Initial kernel scaffold; baseline (speedup 1.0000x reference)
import jax
import jax.numpy as jnp
from jax.experimental import pallas as pl


def kernel(x, edge_index, edges_type, Win, b_in, Wrel, Wself, b_rel, Wu1, b_u1, Wu2, b_u2):
    raise NotImplementedError("write your pallas kernel here")


# speedup vs baseline: 2.2577x; 2.2577x over previous
import functools

import jax
import jax.numpy as jnp
from jax import lax
from jax.experimental import pallas as pl
from jax.experimental.pallas import tpu as pltpu
from jax.experimental.pallas import tpu_sc as plsc

N = 10000
E = 320000
H = 128
R = 16
L = 10

NC = 2
NS = 16
HALFN = N // NC
CHUNK = 128
NCHUNK = -(-E // (NS * CHUNK))
E_PAD = NS * NCHUNK * CHUNK
AGG_ROWS = 5120
ROWS_PER_SUB = AGG_ROWS // NS

BN = 1000
GRID = N // BN


def _sc_body(hall, gidx, dstp, out, idxbuf, dstbuf, rows, aggs, sems):
    cid = lax.axis_index("c")
    sid = lax.axis_index("s")

    def zero_row(r, _):
        for j in range(H // 16):
            rows[0, r, pl.ds(j * 16, 16)] = jnp.zeros((16,), jnp.float32)
        return 0

    lax.fori_loop(0, CHUNK, zero_row, 0)
    base = sid * ROWS_PER_SUB
    off = 0
    while off < ROWS_PER_SUB:
        step = min(CHUNK, ROWS_PER_SUB - off)
        pltpu.sync_copy(rows.at[0, pl.ds(0, step)],
                        aggs.at[pl.ds(base + off, step)])
        off += step

    pltpu.sync_copy(gidx.at[sid], idxbuf)
    pltpu.sync_copy(dstp.at[cid, sid], dstbuf)

    plsc.subcore_barrier()

    pltpu.async_copy(hall.at[idxbuf.at[0]], rows.at[0], sems.at[0])

    def chunk_step(c, _):
        b = lax.rem(c, 2)
        nb = lax.rem(c + 1, 2)
        pltpu.make_async_copy(hall.at[idxbuf.at[c]], rows.at[b],
                              sems.at[b]).wait()

        @pl.when(c + 1 < NCHUNK)
        def _start_next():
            pltpu.async_copy(hall.at[idxbuf.at[c + 1]], rows.at[nb],
                             sems.at[nb])

        pltpu.sync_copy(rows.at[b], aggs.at[dstbuf.at[c]], add=True)
        return 0

    lax.fori_loop(0, NCHUNK, chunk_step, 0)

    plsc.subcore_barrier()

    pltpu.sync_copy(aggs.at[pl.ds(base, ROWS_PER_SUB)],
                    out.at[cid, pl.ds(base, ROWS_PER_SUB)])


@functools.cache
def _sc_scatter():
    return functools.partial(
        pl.kernel,
        mesh=plsc.VectorSubcoreMesh(core_axis_name="c", subcore_axis_name="s"),
        out_type=jax.ShapeDtypeStruct((NC, AGG_ROWS, H), jnp.float32),
        scratch_types=[
            pltpu.VMEM((NCHUNK, CHUNK), jnp.int32),
            pltpu.VMEM((NCHUNK, CHUNK), jnp.int32),
            pltpu.VMEM((2, CHUNK, H), jnp.float32),
            pltpu.VMEM_SHARED((AGG_ROWS, H), jnp.float32),
            pltpu.SemaphoreType.DMA((2,)),
        ],
    )(_sc_body)


def _relu(v):
    return jnp.maximum(v, 0.0)


def _mm(a, b):
    return jnp.dot(a, b, preferred_element_type=jnp.float32)


def _init_body(x, win, b_in, wrel, h_ref, hall_ref):
    h = _relu(_mm(x[...], win[...]) + b_in[0][None, :])
    h_ref[...] = h
    for r in range(R):
        hall_ref[r] = _mm(h, wrel[r])


def _mlp(hb, agg, wself, b_rel, wu1, b_u1, wu2, b_u2):
    mid = agg[0] + _mm(hb, wself[...]) + b_rel[0][None, :]
    z = _relu(_mm(hb, wu1[0:H, :]) + _mm(mid, wu1[H:2 * H, :])
              + b_u1[0][None, :])
    return _relu(_mm(z, wu2[...]) + b_u2[0][None, :])


def _layer_body(h, agg, wself, b_rel, wu1, b_u1, wu2, b_u2, wrel,
                h_ref, hall_ref):
    out = _mlp(h[...], agg, wself, b_rel, wu1, b_u1, wu2, b_u2)
    h_ref[...] = out
    for r in range(R):
        hall_ref[r] = _mm(out, wrel[r])


def _final_body(h, agg, wself, b_rel, wu1, b_u1, wu2, b_u2, h_ref):
    h_ref[...] = _mlp(h[...], agg, wself, b_rel, wu1, b_u1, wu2, b_u2)


def _full(shape):
    return pl.BlockSpec(shape, lambda i: tuple(0 for _ in shape))


_ROWBLK = pl.BlockSpec((BN, H), lambda i: (i, 0))
_HALLBLK = pl.BlockSpec((R, BN, H), lambda i: (0, i, 0))
_AGGBLK = pl.BlockSpec((1, BN, H), lambda i: (i // 5, i % 5, 0))
_HALL_SHAPE = jax.ShapeDtypeStruct((R, N, H), jnp.float32)

_MLP_SPECS = [_full((H, H)), _full((1, H)), _full((2 * H, 2 * H)),
              _full((1, 2 * H)), _full((2 * H, H)), _full((1, H))]

_tc_init = pl.pallas_call(
    _init_body,
    grid=(GRID,),
    in_specs=[_ROWBLK, _full((H, H)), _full((1, H)), _full((R, H, H))],
    out_specs=[_ROWBLK, _HALLBLK],
    out_shape=[jax.ShapeDtypeStruct((N, H), jnp.float32), _HALL_SHAPE],
)

_tc_layer = pl.pallas_call(
    _layer_body,
    grid=(GRID,),
    in_specs=[_ROWBLK, _AGGBLK] + _MLP_SPECS + [_full((R, H, H))],
    out_specs=[_ROWBLK, _HALLBLK],
    out_shape=[jax.ShapeDtypeStruct((N, H), jnp.float32), _HALL_SHAPE],
)

_tc_final = pl.pallas_call(
    _final_body,
    grid=(GRID,),
    in_specs=[_ROWBLK, _AGGBLK] + _MLP_SPECS,
    out_specs=_ROWBLK,
    out_shape=jax.ShapeDtypeStruct((N, H), jnp.float32),
)


def kernel(x, edge_index, edges_type, Win, b_in, Wrel, Wself, b_rel,
           Wu1, b_u1, Wu2, b_u2):
    src = edge_index[0].astype(jnp.int32)
    dst = edge_index[1].astype(jnp.int32)
    et = edges_type.astype(jnp.int32)

    gidx = et * N + src
    pad = E_PAD - E
    gidx_p = jnp.concatenate([gidx, jnp.zeros((pad,), jnp.int32)])
    gidx_p = gidx_p.reshape(NS, NCHUNK, CHUNK)
    dstl = []
    for c in range(NC):
        loc = dst - c * HALFN
        loc = jnp.where((loc >= 0) & (loc < HALFN), loc, HALFN)
        dstl.append(jnp.concatenate([loc, jnp.full((pad,), HALFN, jnp.int32)]))
    dst_p = jnp.stack(dstl).reshape(NC, NS, NCHUNK, CHUNK)

    h, hall = _tc_init(x, Win, b_in.reshape(1, H), Wrel[0])
    for l in range(L):
        agg = _sc_scatter()(hall.reshape(R * N, H), gidx_p, dst_p)
        args = (h, agg, Wself[l], b_rel[l].reshape(1, H), Wu1[l],
                b_u1[l].reshape(1, 2 * H), Wu2[l], b_u2[l].reshape(1, H))
        if l < L - 1:
            h, hall = _tc_layer(*args, Wrel[l + 1])
        else:
            h = _tc_final(*args)
    return h
